# baseline (speedup 1.0000x reference)
import functools

import jax
import jax.numpy as jnp
from jax import lax
from jax.experimental import pallas as pl
from jax.experimental.pallas import tpu as pltpu

CH = 32


def _body(row_ref, col_ref, out_hbm, buf0, buf1, sem0, sem1, *, B, e, H, W):
    n_dim = 2 * e
    HW = H * W
    lane = lax.broadcasted_iota(jnp.int32, (CH, HW), 1)
    sub = lax.broadcasted_iota(jnp.int32, (CH, HW), 0)
    P = (lane % W == sub).astype(jnp.float32)
    Q = (lane // W == sub).astype(jnp.float32)

    bufs = (buf0, buf1)
    sems = (sem0, sem1)
    n_chunks = n_dim // CH
    half = e // CH
    for k in range(n_chunks):
        buf, sem = bufs[k % 2], sems[k % 2]
        if k >= 2:
            pltpu.make_async_copy(
                buf, out_hbm.at[:, pl.ds((k - 2) * CH, CH), :], sem
            ).wait()
        if k < half:
            tblk = col_ref[:W, pl.ds(k * CH, CH)]
            mask = P
        else:
            tblk = row_ref[:H, pl.ds(k * CH - e, CH)]
            mask = Q
        content = lax.dot_general(
            tblk, mask, (((0,), (0,)), ((), ())),
            preferred_element_type=jnp.float32,
        )
        buf[...] = jnp.broadcast_to(content[None], (B, CH, HW))
        pltpu.make_async_copy(
            buf, out_hbm.at[:, pl.ds(k * CH, CH), :], sem
        ).start()
    for k in (n_chunks - 2, n_chunks - 1):
        pltpu.make_async_copy(
            bufs[k % 2], out_hbm.at[:, pl.ds(k * CH, CH), :], sems[k % 2]
        ).wait()


def kernel(x, row_embed, col_embed):
    B = x.shape[0]
    H, W = x.shape[-2], x.shape[-1]
    e = row_embed.shape[1]
    n_dim = 2 * e
    out = pl.pallas_call(
        functools.partial(_body, B=B, e=e, H=H, W=W),
        in_specs=[
            pl.BlockSpec(memory_space=pltpu.MemorySpace.VMEM),
            pl.BlockSpec(memory_space=pltpu.MemorySpace.VMEM),
        ],
        out_specs=pl.BlockSpec(memory_space=pltpu.MemorySpace.HBM),
        out_shape=jax.ShapeDtypeStruct((B, n_dim, H * W), row_embed.dtype),
        scratch_shapes=[
            pltpu.VMEM((B, CH, H * W), row_embed.dtype),
            pltpu.VMEM((B, CH, H * W), row_embed.dtype),
            pltpu.SemaphoreType.DMA,
            pltpu.SemaphoreType.DMA,
        ],
    )(row_embed, col_embed)
    return out.reshape(B, n_dim, H, W)
